# baseline (speedup 1.0000x reference)
import functools

import jax
import jax.numpy as jnp
from jax import lax
from jax.experimental import pallas as pl
from jax.experimental.pallas import tpu as pltpu
from jax.experimental.pallas import tpu_sc as plsc

_NC = 2
_NS = 16
_NW = _NC * _NS
_CH = 128


def _sc_gather4(user, item, eug, eig, eum, eim):
    B = user.shape[0]
    D = eug.shape[1]
    bpw = B // _NW
    nch = bpw // _CH

    mesh = plsc.VectorSubcoreMesh(core_axis_name="c", subcore_axis_name="s")

    @functools.partial(
        pl.kernel,
        mesh=mesh,
        compiler_params=pltpu.CompilerParams(use_tc_tiling_on_sc=False),
        out_type=[jax.ShapeDtypeStruct((B, D), jnp.float32)] * 4,
        scratch_types=[
            pltpu.VMEM((_CH,), jnp.int32),
            pltpu.VMEM((_CH, D), jnp.float32),
            pltpu.SemaphoreType.DMA,
        ],
    )
    def k(user_h, item_h, eug_h, eig_h, eum_h, eim_h,
          o_ug, o_ig, o_um, o_im, idx_v, rows_v, sem):
        wid = lax.axis_index("s") * _NC + lax.axis_index("c")
        base = wid * bpw
        for j in range(nch):
            off = base + j * _CH
            pltpu.sync_copy(user_h.at[pl.ds(off, _CH)], idx_v)
            pltpu.async_copy(eug_h.at[idx_v], rows_v, sem).wait()
            pltpu.sync_copy(rows_v, o_ug.at[pl.ds(off, _CH)])
            pltpu.async_copy(eum_h.at[idx_v], rows_v, sem).wait()
            pltpu.sync_copy(rows_v, o_um.at[pl.ds(off, _CH)])
            pltpu.sync_copy(item_h.at[pl.ds(off, _CH)], idx_v)
            pltpu.async_copy(eig_h.at[idx_v], rows_v, sem).wait()
            pltpu.sync_copy(rows_v, o_ig.at[pl.ds(off, _CH)])
            pltpu.async_copy(eim_h.at[idx_v], rows_v, sem).wait()
            pltpu.sync_copy(rows_v, o_im.at[pl.ds(off, _CH)])

    return k(user, item, eug, eig, eum, eim)


def _tc_dense(pu_g, qi_g, pu_m, qi_m, w1u, w1i, b1, w2, b2, w3, b3,
              whg, whl, bh):
    B, D = pu_g.shape
    H1 = w1u.shape[1]
    H2 = w2.shape[1]
    H3 = w3.shape[1]
    BB = 2048
    grid = B // BB

    def body(pu_g_r, qi_g_r, pu_m_r, qi_m_r, w1u_r, w1i_r, b1_r, w2_r, b2_r,
             w3_r, b3_r, whg_r, whl_r, bh_r, out_r):
        gmf = pu_g_r[...] * qi_g_r[...]
        l1 = jnp.maximum(
            jnp.dot(pu_m_r[...], w1u_r[...], preferred_element_type=jnp.float32)
            + jnp.dot(qi_m_r[...], w1i_r[...], preferred_element_type=jnp.float32)
            + b1_r[...], 0.0)
        l2 = jnp.maximum(
            jnp.dot(l1, w2_r[...], preferred_element_type=jnp.float32)
            + b2_r[...], 0.0)
        l3 = jnp.maximum(
            jnp.dot(l2, w3_r[...], preferred_element_type=jnp.float32)
            + b3_r[...], 0.0)
        s = (jnp.sum(gmf * whg_r[...], axis=1, keepdims=True)
             + jnp.sum(l3 * whl_r[...], axis=1, keepdims=True)
             + bh_r[...])
        out_r[...] = 1.0 / (1.0 + jnp.exp(-s))

    emb_spec = pl.BlockSpec((BB, D), lambda i: (i, 0))
    full = lambda a: pl.BlockSpec(a.shape, lambda i: (0,) * a.ndim)
    return pl.pallas_call(
        body,
        grid=(grid,),
        in_specs=[emb_spec, emb_spec, emb_spec, emb_spec,
                  full(w1u), full(w1i), full(b1), full(w2), full(b2),
                  full(w3), full(b3), full(whg), full(whl), full(bh)],
        out_specs=pl.BlockSpec((BB, 1), lambda i: (i, 0)),
        out_shape=jax.ShapeDtypeStruct((B, 1), jnp.float32),
    )(pu_g, qi_g, pu_m, qi_m, w1u, w1i, b1, w2, b2, w3, b3, whg, whl, bh)


def kernel(user, item, embed_u_gmf, embed_i_gmf, embed_u_mlp, embed_i_mlp,
           W1, b1, W2, b2, W3, b3, WH, bH):
    D = embed_u_gmf.shape[1]
    user = user.astype(jnp.int32)
    item = item.astype(jnp.int32)
    pu_g, qi_g, pu_m, qi_m = _sc_gather4(
        user, item, embed_u_gmf, embed_i_gmf, embed_u_mlp, embed_i_mlp)
    w1u = W1[:D]
    w1i = W1[D:]
    whg = WH[:D].T
    whl = WH[D:].T
    return _tc_dense(pu_g, qi_g, pu_m, qi_m, w1u, w1i,
                     b1.reshape(1, -1), W2, b2.reshape(1, -1),
                     W3, b3.reshape(1, -1), whg, whl, bH.reshape(1, 1))


# speedup vs baseline: 1.0103x; 1.0103x over previous
import functools

import jax
import jax.numpy as jnp
from jax import lax
from jax.experimental import pallas as pl
from jax.experimental.pallas import tpu as pltpu
from jax.experimental.pallas import tpu_sc as plsc

_NC = 2
_NS = 16
_NW = _NC * _NS
_CH = 128


def _sc_gather4(u2, i2, tug, tig, tum, tim):
    nrow, ncol = u2.shape
    B = nrow * ncol
    W = tug.shape[1]
    bpw = B // _NW
    nch = bpw // _CH
    rows_per_w = bpw // ncol

    mesh = plsc.VectorSubcoreMesh(core_axis_name="c", subcore_axis_name="s")

    @functools.partial(
        pl.kernel,
        mesh=mesh,
        out_type=[jax.ShapeDtypeStruct((B, W), jnp.float32)] * 4,
        scratch_types=[
            pltpu.VMEM((rows_per_w, ncol), jnp.int32),
            pltpu.VMEM((rows_per_w, ncol), jnp.int32),
            pltpu.VMEM((_CH, W), jnp.float32),
            pltpu.VMEM((_CH, W), jnp.float32),
            pltpu.SemaphoreType.DMA,
            pltpu.SemaphoreType.DMA,
            pltpu.SemaphoreType.DMA,
            pltpu.SemaphoreType.DMA,
        ],
    )
    def k(u2_h, i2_h, tug_h, tig_h, tum_h, tim_h,
          o_ug, o_ig, o_um, o_im,
          idx_u, idx_i, buf0, buf1, sg0, sg1, sw0, sw1):
        wid = lax.axis_index("s") * _NC + lax.axis_index("c")
        base = wid * bpw
        pltpu.sync_copy(u2_h.at[pl.ds(wid * rows_per_w, rows_per_w)], idx_u)
        pltpu.sync_copy(i2_h.at[pl.ds(wid * rows_per_w, rows_per_w)], idx_i)

        tasks = []
        for tbl, out, idxv in ((tug_h, o_ug, idx_u), (tum_h, o_um, idx_u),
                               (tig_h, o_ig, idx_i), (tim_h, o_im, idx_i)):
            for j in range(nch):
                tasks.append((tbl, out, idxv, j))

        bufs = (buf0, buf1)
        sgs = (sg0, sg1)
        sws = (sw0, sw1)
        gathers = [None, None]
        writes = [None, None]

        def fire_gather(t):
            tbl, _, idxv, j = tasks[t]
            p = t & 1
            gathers[p] = pltpu.async_copy(tbl.at[idxv.at[j]], bufs[p], sgs[p])

        fire_gather(0)
        for t in range(len(tasks)):
            p = t & 1
            if t + 1 < len(tasks):
                q = (t + 1) & 1
                if writes[q] is not None:
                    writes[q].wait()
                    writes[q] = None
                fire_gather(t + 1)
            gathers[p].wait()
            _, out, _, j = tasks[t]
            writes[p] = pltpu.async_copy(
                bufs[p], out.at[pl.ds(base + j * _CH, _CH)], sws[p])
        for p in (0, 1):
            if writes[p] is not None:
                writes[p].wait()

    return k(u2, i2, tug, tig, tum, tim)


def _tc_dense(r_ug, r_ig, r_um, r_im, up, ip, w1u, w1i, b1, w2, b2, w3, b3,
              whg, whl, bh):
    B, W = r_ug.shape
    D = W // 2
    BB = 2048
    grid = B // BB

    def body(r_ug_r, r_ig_r, r_um_r, r_im_r, up_r, ip_r,
             w1u_r, w1i_r, b1_r, w2_r, b2_r, w3_r, b3_r,
             whg_r, whl_r, bh_r, out_r):
        mu = up_r[...] == 0
        mi = ip_r[...] == 0
        rug = r_ug_r[...]
        rig = r_ig_r[...]
        rum = r_um_r[...]
        rim = r_im_r[...]
        pu_g = jnp.where(mu, rug[:, :D], rug[:, D:])
        qi_g = jnp.where(mi, rig[:, :D], rig[:, D:])
        pu_m = jnp.where(mu, rum[:, :D], rum[:, D:])
        qi_m = jnp.where(mi, rim[:, :D], rim[:, D:])
        gmf = pu_g * qi_g
        l1 = jnp.maximum(
            jnp.dot(pu_m, w1u_r[...], preferred_element_type=jnp.float32)
            + jnp.dot(qi_m, w1i_r[...], preferred_element_type=jnp.float32)
            + b1_r[...], 0.0)
        l2 = jnp.maximum(
            jnp.dot(l1, w2_r[...], preferred_element_type=jnp.float32)
            + b2_r[...], 0.0)
        l3 = jnp.maximum(
            jnp.dot(l2, w3_r[...], preferred_element_type=jnp.float32)
            + b3_r[...], 0.0)
        s = (jnp.sum(gmf * whg_r[...], axis=1, keepdims=True)
             + jnp.sum(l3 * whl_r[...], axis=1, keepdims=True)
             + bh_r[...])
        out_r[...] = 1.0 / (1.0 + jnp.exp(-s))

    row_spec = pl.BlockSpec((BB, W), lambda i: (i, 0))
    par_spec = pl.BlockSpec((BB, 1), lambda i: (i, 0))
    full = lambda a: pl.BlockSpec(a.shape, lambda i: (0,) * a.ndim)
    return pl.pallas_call(
        body,
        grid=(grid,),
        in_specs=[row_spec, row_spec, row_spec, row_spec, par_spec, par_spec,
                  full(w1u), full(w1i), full(b1), full(w2), full(b2),
                  full(w3), full(b3), full(whg), full(whl), full(bh)],
        out_specs=pl.BlockSpec((BB, 1), lambda i: (i, 0)),
        out_shape=jax.ShapeDtypeStruct((B, 1), jnp.float32),
    )(r_ug, r_ig, r_um, r_im, up, ip, w1u, w1i, b1, w2, b2, w3, b3,
      whg, whl, bh)


def kernel(user, item, embed_u_gmf, embed_i_gmf, embed_u_mlp, embed_i_mlp,
           W1, b1, W2, b2, W3, b3, WH, bH):
    B = user.shape[0]
    D = embed_u_gmf.shape[1]
    user = user.astype(jnp.int32)
    item = item.astype(jnp.int32)
    u2 = (user >> 1).reshape(B // 128, 128)
    i2 = (item >> 1).reshape(B // 128, 128)
    up = (user & 1).reshape(B, 1)
    ip = (item & 1).reshape(B, 1)
    V = embed_u_gmf.shape[0]
    tug = embed_u_gmf.reshape(V // 2, 2 * D)
    tig = embed_i_gmf.reshape(V // 2, 2 * D)
    tum = embed_u_mlp.reshape(V // 2, 2 * D)
    tim = embed_i_mlp.reshape(V // 2, 2 * D)
    r_ug, r_ig, r_um, r_im = _sc_gather4(u2, i2, tug, tig, tum, tim)
    w1u = W1[:D]
    w1i = W1[D:]
    whg = WH[:D].T
    whl = WH[D:].T
    return _tc_dense(r_ug, r_ig, r_um, r_im, up, ip, w1u, w1i,
                     b1.reshape(1, -1), W2, b2.reshape(1, -1),
                     W3, b3.reshape(1, -1), whg, whl, bH.reshape(1, 1))
